# Initial kernel scaffold; baseline (speedup 1.0000x reference)
import jax
import jax.numpy as jnp
from jax.experimental import pallas as pl


def kernel(x, pos_edge_index, neg_edge_index, a1_pos, a1_neg, W1_pos, b1_pos, W1_neg, b1_neg, a2_pos, a2_neg, W2_pos, b2_pos, W2_neg, b2_neg):
    raise NotImplementedError("write your pallas kernel here")


# speedup vs baseline: 47.1364x; 47.1364x over previous
import functools

import jax
import jax.numpy as jnp
from jax import lax
from jax.experimental import pallas as pl
from jax.experimental.pallas import tpu as pltpu
from jax.experimental.pallas import tpu_sc as plsc

N = 50000
DIN = 64
HID = 32
NSC = 2
NT = 16
N_PAD = 50176
STRIPE = N_PAD // NT
HSTRIPE = STRIPE // 2
KS = 512
E_T_POS = 18944
E_T_NEG = 6656
B_TC = 1792
RPB = KS // 128


def _prep_edges(ei, e_tile):
    e_tot = NSC * NT * e_tile
    e = ei.shape[1]
    pad = e_tot - e
    pad_src = jnp.arange(pad, dtype=jnp.int32) % N
    pad_dst = N + jnp.arange(pad, dtype=jnp.int32) % (N_PAD - N)
    src = jnp.concatenate([ei[0], pad_src])
    dst = jnp.concatenate([ei[1], pad_dst])
    return src.reshape(-1, 128), dst.reshape(-1, 128)


def _make_sc_agg(e_tile, n_h):
    nblk = e_tile // KS
    rows_per_tile = e_tile // 128
    mesh = plsc.VectorSubcoreMesh(core_axis_name="c", subcore_axis_name="s")

    def body(m_hbm, u_hbm, v_hbm, *rest):
        zeros16 = jnp.zeros((16,), jnp.float32)
        h_hbms = rest[:n_h]
        (src_hbm, dst_hbm, out_hbm, s_hbm,
         u_sp, v_sp, s_sp, out_sp, semf, sems) = rest[n_h:]
        c = lax.axis_index("c")
        s = lax.axis_index("s")
        tile_rows = (c * NT + s) * rows_per_tile

        def phases(idx_s, idx_d, ub, vb, wb, rows_b, zb1, zb2, mv):
            st = pl.ds(s * STRIPE, STRIPE)
            for h in range(2):
                sth = pl.ds(s * STRIPE + h * HSTRIPE, HSTRIPE)
                pltpu.sync_copy(u_hbm.at[sth], zb1)
                pltpu.sync_copy(zb1, u_sp.at[sth])
                pltpu.sync_copy(v_hbm.at[sth], zb1)
                pltpu.sync_copy(zb1, v_sp.at[sth])

            pltpu.sync_copy(m_hbm, mv)
            mvec = mv[pl.ds(0, 16)]

            def z1(i, _):
                zb1[pl.ds(i * 16, 16)] = zeros16
                return 0
            lax.fori_loop(0, HSTRIPE // 16, z1, 0, unroll=8)

            def z2(i, _):
                zb2[i, pl.ds(0, 16)] = zeros16
                zb2[i, pl.ds(16, 16)] = zeros16
                return 0
            lax.fori_loop(0, 16, z2, 0, unroll=8)

            for h in range(2):
                pltpu.sync_copy(zb1, s_sp.at[pl.ds(s * STRIPE + h * HSTRIPE,
                                                   HSTRIPE)])
            plsc.subcore_barrier()

            for hi in range(n_h):
                h_hbm = h_hbms[hi]

                def z3(i, _):
                    pltpu.sync_copy(zb2, out_sp.at[pl.ds(s * STRIPE + i * 16, 16)])
                    return 0
                lax.fori_loop(0, STRIPE // 16, z3, 0)
                plsc.subcore_barrier()

                def blk(b, _):
                    row0 = tile_rows + b * RPB
                    pltpu.sync_copy(src_hbm.at[pl.ds(row0, RPB)], idx_s)
                    pltpu.sync_copy(dst_hbm.at[pl.ds(row0, RPB)], idx_d)
                    cpf = [pltpu.async_copy(h_hbm.at[idx_s.at[r]],
                                            rows_b.at[pl.ds(r * 128, 128)], semf)
                           for r in range(RPB)]
                    cps = [pltpu.async_copy(u_sp.at[idx_s.at[r]], ub.at[r], sems)
                           for r in range(RPB)]
                    cps += [pltpu.async_copy(v_sp.at[idx_d.at[r]], vb.at[r], sems)
                            for r in range(RPB)]
                    for cp in cps:
                        cp.wait()
                    for r in range(RPB):
                        for q in range(8):
                            e = (ub[r, pl.ds(q * 16, 16)]
                                 + vb[r, pl.ds(q * 16, 16)])
                            e = jnp.where(e >= 0.0, e, 0.2 * e)
                            wb[r, pl.ds(q * 16, 16)] = jnp.exp(e - mvec)
                    if hi == 0:
                        for r in range(RPB):
                            pltpu.sync_copy(wb.at[r], s_sp.at[idx_d.at[r]],
                                            add=True)
                    for cp in cpf:
                        cp.wait()

                    def sc1(j, _):
                        w16 = wb[j // 8, pl.ds((j % 8) * 16, 16)]
                        for k in range(16):
                            rr = j * 16 + k
                            w_s = w16[k]
                            rows_b[rr, pl.ds(0, 16)] = rows_b[rr, pl.ds(0, 16)] * w_s
                            rows_b[rr, pl.ds(16, 16)] = rows_b[rr, pl.ds(16, 16)] * w_s
                        return 0
                    lax.fori_loop(0, KS // 16, sc1, 0)

                    for r in range(RPB):
                        pltpu.sync_copy(rows_b.at[pl.ds(r * 128, 128)],
                                        out_sp.at[idx_d.at[r]], add=True)
                    return 0

                lax.fori_loop(0, nblk, blk, 0)
                plsc.subcore_barrier()
                for i in range(7):
                    ck = pl.ds(s * STRIPE + i * 448, 448)
                    pltpu.sync_copy(out_sp.at[ck], rows_b.at[pl.ds(0, 448)])
                    pltpu.sync_copy(rows_b.at[pl.ds(0, 448)],
                                    out_hbm.at[hi, c, ck])
                plsc.subcore_barrier()

            for h in range(2):
                sth = pl.ds(s * STRIPE + h * HSTRIPE, HSTRIPE)
                pltpu.sync_copy(s_sp.at[sth], zb1)
                pltpu.sync_copy(zb1, s_hbm.at[c, sth])

        pl.run_scoped(
            phases,
            pltpu.VMEM((RPB, 128), jnp.int32),
            pltpu.VMEM((RPB, 128), jnp.int32),
            pltpu.VMEM((RPB, 128), jnp.float32),
            pltpu.VMEM((RPB, 128), jnp.float32),
            pltpu.VMEM((RPB, 128), jnp.float32),
            pltpu.VMEM((KS, HID), jnp.float32),
            pltpu.VMEM((HSTRIPE,), jnp.float32),
            pltpu.VMEM((16, HID), jnp.float32),
            pltpu.VMEM((16,), jnp.float32),
        )

    return pl.kernel(
        body,
        out_type=(
            jax.ShapeDtypeStruct((n_h, NSC, N_PAD, HID), jnp.float32),
            jax.ShapeDtypeStruct((NSC, N_PAD), jnp.float32),
        ),
        mesh=mesh,
        compiler_params=pltpu.CompilerParams(
            needs_layout_passes=False, use_tc_tiling_on_sc=False),
        scratch_types=[
            pltpu.VMEM_SHARED((N_PAD,), jnp.float32),
            pltpu.VMEM_SHARED((N_PAD,), jnp.float32),
            pltpu.VMEM_SHARED((N_PAD,), jnp.float32),
            pltpu.VMEM_SHARED((N_PAD, HID), jnp.float32),
            pltpu.SemaphoreType.DMA,
            pltpu.SemaphoreType.DMA,
        ],
    )


_sc_agg_pos2 = _make_sc_agg(E_T_POS, 2)
_sc_agg_neg2 = _make_sc_agg(E_T_NEG, 2)
_sc_agg_pos1 = _make_sc_agg(E_T_POS, 1)
_sc_agg_neg1 = _make_sc_agg(E_T_NEG, 1)


def _tc_scores_body(x_ref, a_ref, out_ref, mx_ref):
    i = pl.program_id(0)
    blk = lax.dot_general(
        a_ref[...], x_ref[...], (((1,), (1,)), ((), ())),
        preferred_element_type=jnp.float32)
    out_ref[...] = blk
    bm = jnp.broadcast_to(jnp.max(blk, axis=1)[:, None], (8, 128))

    @pl.when(i == 0)
    def _():
        mx_ref[...] = bm

    @pl.when(i > 0)
    def _():
        mx_ref[...] = jnp.maximum(mx_ref[...], bm)


def _tc_scores(x, a_mat):
    return pl.pallas_call(
        _tc_scores_body,
        grid=(N_PAD // B_TC,),
        in_specs=[pl.BlockSpec((B_TC, DIN), lambda i: (i, 0)),
                  pl.BlockSpec((8, DIN), lambda i: (0, 0))],
        out_specs=[pl.BlockSpec((8, B_TC), lambda i: (0, i)),
                   pl.BlockSpec((8, 128), lambda i: (0, 0))],
        out_shape=[jax.ShapeDtypeStruct((8, N_PAD), jnp.float32),
                   jax.ShapeDtypeStruct((8, 128), jnp.float32)],
    )(x, a_mat)


def _combine(p_ref, s_ref, n_h):
    den = s_ref[0] + s_ref[1]
    inv = 1.0 / (den + 1e-16)
    parts = [(p_ref[h, 0] + p_ref[h, 1]) * inv[:, None] for h in range(n_h)]
    return jnp.concatenate(parts, axis=1) if n_h > 1 else parts[0]


def _tc_layer1_body(x_ref, pp_ref, sp_ref, pn_ref, sn_ref,
                    w1p_ref, b1p_ref, w1n_ref, b1n_ref, wuv_ref,
                    zp_ref, zn_ref, uv2_ref, mx_ref):
    i = pl.program_id(0)
    x = x_ref[...]
    agg_p = _combine(pp_ref, sp_ref, 2)
    agg_n = _combine(pn_ref, sn_ref, 2)
    hp = jnp.concatenate([agg_p, x], axis=1) @ w1p_ref[...] + b1p_ref[...]
    hn = jnp.concatenate([agg_n, x], axis=1) @ w1n_ref[...] + b1n_ref[...]
    zp = jnp.tanh(hp)
    zn = jnp.tanh(hn)
    zp_ref[...] = zp
    zn_ref[...] = zn
    uv2 = lax.dot_general(
        wuv_ref[...], jnp.concatenate([zp, zn], axis=1),
        (((1,), (1,)), ((), ())), preferred_element_type=jnp.float32)
    uv2_ref[...] = uv2
    bm = jnp.broadcast_to(jnp.max(uv2, axis=1)[:, None], (8, 128))

    @pl.when(i == 0)
    def _():
        mx_ref[...] = bm

    @pl.when(i > 0)
    def _():
        mx_ref[...] = jnp.maximum(mx_ref[...], bm)


def _tc_layer1(x, pp, sp, pn, sn, w1p, b1p, w1n, b1n, wuv):
    part2 = pl.BlockSpec((2, NSC, B_TC, HID), lambda i: (0, 0, i, 0))
    sspec = pl.BlockSpec((NSC, B_TC), lambda i: (0, i))
    return pl.pallas_call(
        _tc_layer1_body,
        grid=(N_PAD // B_TC,),
        in_specs=[pl.BlockSpec((B_TC, DIN), lambda i: (i, 0)),
                  part2, sspec, part2, sspec,
                  pl.BlockSpec((2 * DIN, HID), lambda i: (0, 0)),
                  pl.BlockSpec((HID,), lambda i: (0,)),
                  pl.BlockSpec((2 * DIN, HID), lambda i: (0, 0)),
                  pl.BlockSpec((HID,), lambda i: (0,)),
                  pl.BlockSpec((8, 2 * HID), lambda i: (0, 0))],
        out_specs=[pl.BlockSpec((B_TC, HID), lambda i: (i, 0)),
                   pl.BlockSpec((B_TC, HID), lambda i: (i, 0)),
                   pl.BlockSpec((8, B_TC), lambda i: (0, i)),
                   pl.BlockSpec((8, 128), lambda i: (0, 0))],
        out_shape=[jax.ShapeDtypeStruct((N_PAD, HID), jnp.float32),
                   jax.ShapeDtypeStruct((N_PAD, HID), jnp.float32),
                   jax.ShapeDtypeStruct((8, N_PAD), jnp.float32),
                   jax.ShapeDtypeStruct((8, 128), jnp.float32)],
    )(x, pp, sp, pn, sn, w1p, b1p, w1n, b1n, wuv)


def _tc_layer2_body(zp_ref, zn_ref,
                    ppp_ref, spp_ref, pnn_ref, snn_ref,
                    pnp_ref, snp_ref, ppn_ref, spn_ref,
                    w2p_ref, b2p_ref, w2n_ref, b2n_ref, out_ref):
    agg_pp = _combine(ppp_ref, spp_ref, 1)
    agg_nn = _combine(pnn_ref, snn_ref, 1)
    agg_np = _combine(pnp_ref, snp_ref, 1)
    agg_pn = _combine(ppn_ref, spn_ref, 1)
    op = (jnp.concatenate([agg_pp, agg_nn, zp_ref[...]], axis=1)
          @ w2p_ref[...] + b2p_ref[...])
    on = (jnp.concatenate([agg_np, agg_pn, zn_ref[...]], axis=1)
          @ w2n_ref[...] + b2n_ref[...])
    out_ref[...] = jnp.tanh(jnp.concatenate([op, on], axis=1))


def _tc_layer2(zp, zn, aggs, w2p, b2p, w2n, b2n):
    part1 = pl.BlockSpec((1, NSC, B_TC, HID), lambda i: (0, 0, i, 0))
    sspec = pl.BlockSpec((NSC, B_TC), lambda i: (0, i))
    zspec = pl.BlockSpec((B_TC, HID), lambda i: (i, 0))
    agg_ops = []
    agg_specs = []
    for (p, sv) in aggs:
        agg_ops += [p, sv]
        agg_specs += [part1, sspec]
    return pl.pallas_call(
        _tc_layer2_body,
        grid=(N_PAD // B_TC,),
        in_specs=[zspec, zspec] + agg_specs +
                 [pl.BlockSpec((3 * HID, HID), lambda i: (0, 0)),
                  pl.BlockSpec((HID,), lambda i: (0,)),
                  pl.BlockSpec((3 * HID, HID), lambda i: (0, 0)),
                  pl.BlockSpec((HID,), lambda i: (0,))],
        out_specs=pl.BlockSpec((B_TC, DIN), lambda i: (i, 0)),
        out_shape=jax.ShapeDtypeStruct((N_PAD, DIN), jnp.float32),
    )(zp, zn, *agg_ops, w2p, b2p, w2n, b2n)


def kernel(x, pos_edge_index, neg_edge_index,
           a1_pos, a1_neg, W1_pos, b1_pos, W1_neg, b1_neg,
           a2_pos, a2_neg, W2_pos, b2_pos, W2_neg, b2_neg):
    zeros32 = jnp.zeros((HID,), jnp.float32)

    psrc, pdst = _prep_edges(pos_edge_index, E_T_POS)
    nsrc, ndst = _prep_edges(neg_edge_index, E_T_NEG)
    x0 = x[:, :HID]
    x1 = x[:, HID:]
    a1_mat = jnp.stack([a1_pos[:DIN], a1_pos[DIN:], a1_neg[:DIN], a1_neg[DIN:],
                        jnp.zeros((DIN,), jnp.float32), jnp.zeros((DIN,), jnp.float32),
                        jnp.zeros((DIN,), jnp.float32), jnp.zeros((DIN,), jnp.float32)])
    wuv = jnp.stack([
        jnp.concatenate([a2_pos[:HID], zeros32]),
        jnp.concatenate([a2_pos[HID:], zeros32]),
        jnp.concatenate([zeros32, a2_neg[:HID]]),
        jnp.concatenate([zeros32, a2_neg[HID:]]),
        jnp.concatenate([zeros32, a2_pos[:HID]]),
        jnp.concatenate([zeros32, a2_pos[HID:]]),
        jnp.concatenate([a2_neg[:HID], zeros32]),
        jnp.concatenate([a2_neg[HID:], zeros32]),
    ])

    xp = jnp.zeros((N_PAD, DIN), jnp.float32).at[:N].set(x)
    uv1, mx1 = _tc_scores(xp, a1_mat)
    m1p = mx1[0, :16] + mx1[1, :16]
    m1n = mx1[2, :16] + mx1[3, :16]
    pp, sp = _sc_agg_pos2(m1p, uv1[0], uv1[1], x0, x1, psrc, pdst)
    pn, sn = _sc_agg_neg2(m1n, uv1[2], uv1[3], x0, x1, nsrc, ndst)
    zp, zn, uv2, mx2 = _tc_layer1(xp, pp, sp, pn, sn,
                                  W1_pos, b1_pos, W1_neg, b1_neg, wuv)

    m_pp = mx2[0, :16] + mx2[1, :16]
    m_nn = mx2[2, :16] + mx2[3, :16]
    m_np = mx2[4, :16] + mx2[5, :16]
    m_pn = mx2[6, :16] + mx2[7, :16]
    agg_pp = _sc_agg_pos1(m_pp, uv2[0], uv2[1], zp, psrc, pdst)
    agg_nn = _sc_agg_neg1(m_nn, uv2[2], uv2[3], zn, nsrc, ndst)
    agg_np = _sc_agg_pos1(m_np, uv2[4], uv2[5], zn, psrc, pdst)
    agg_pn = _sc_agg_neg1(m_pn, uv2[6], uv2[7], zp, nsrc, ndst)
    out = _tc_layer2(zp, zn, [agg_pp, agg_nn, agg_np, agg_pn],
                     W2_pos, b2_pos, W2_neg, b2_neg)
    return out[:N]


# speedup vs baseline: 47.8861x; 1.0159x over previous
import functools

import jax
import jax.numpy as jnp
from jax import lax
from jax.experimental import pallas as pl
from jax.experimental.pallas import tpu as pltpu
from jax.experimental.pallas import tpu_sc as plsc

N = 50000
DIN = 64
HID = 32
NSC = 2
NT = 16
N_PAD = 50176
STRIPE = N_PAD // NT
HSTRIPE = STRIPE // 2
KS = 512
E_T_POS = 18944
E_T_NEG = 6656
B_TC = 1792
RPB = KS // 128


def _prep_edges(ei, e_tile):
    e_tot = NSC * NT * e_tile
    e = ei.shape[1]
    pad = e_tot - e
    pad_src = jnp.arange(pad, dtype=jnp.int32) % N
    pad_dst = N + jnp.arange(pad, dtype=jnp.int32) % (N_PAD - N)
    src = jnp.concatenate([ei[0], pad_src])
    dst = jnp.concatenate([ei[1], pad_dst])
    return src.reshape(-1, 128), dst.reshape(-1, 128)


def _make_sc_agg(e_tile, n_h):
    nblk = e_tile // KS
    rows_per_tile = e_tile // 128
    mesh = plsc.VectorSubcoreMesh(core_axis_name="c", subcore_axis_name="s")

    def body(m_hbm, u_hbm, v_hbm, *rest):
        zeros16 = jnp.zeros((16,), jnp.float32)
        h_hbms = rest[:n_h]
        (src_hbm, dst_hbm, out_hbm, s_hbm,
         u_sp, v_sp, s_sp, out_sp, semf, sems) = rest[n_h:]
        c = lax.axis_index("c")
        s = lax.axis_index("s")
        tile_rows = (c * NT + s) * rows_per_tile

        def phases(idx_s, idx_d, ub, vb, wb, rows_b, zb1, zb2, mv):
            st = pl.ds(s * STRIPE, STRIPE)
            for h in range(2):
                sth = pl.ds(s * STRIPE + h * HSTRIPE, HSTRIPE)
                pltpu.sync_copy(u_hbm.at[sth], zb1)
                pltpu.sync_copy(zb1, u_sp.at[sth])
                pltpu.sync_copy(v_hbm.at[sth], zb1)
                pltpu.sync_copy(zb1, v_sp.at[sth])

            pltpu.sync_copy(m_hbm, mv)
            mvec = mv[pl.ds(0, 16)]

            def z1(i, _):
                zb1[pl.ds(i * 16, 16)] = zeros16
                return 0
            lax.fori_loop(0, HSTRIPE // 16, z1, 0, unroll=8)

            def z2(i, _):
                zb2[i, pl.ds(0, 16)] = zeros16
                zb2[i, pl.ds(16, 16)] = zeros16
                return 0
            lax.fori_loop(0, 16, z2, 0, unroll=8)

            for h in range(2):
                pltpu.sync_copy(zb1, s_sp.at[pl.ds(s * STRIPE + h * HSTRIPE,
                                                   HSTRIPE)])
            plsc.subcore_barrier()

            for hi in range(n_h):
                h_hbm = h_hbms[hi]

                def z3(i, _):
                    pltpu.sync_copy(zb2, out_sp.at[pl.ds(s * STRIPE + i * 16, 16)])
                    return 0
                lax.fori_loop(0, STRIPE // 16, z3, 0)
                plsc.subcore_barrier()

                def blk(b, _):
                    row0 = tile_rows + b * RPB
                    pltpu.sync_copy(src_hbm.at[pl.ds(row0, RPB)], idx_s)
                    pltpu.sync_copy(dst_hbm.at[pl.ds(row0, RPB)], idx_d)
                    cpf = [pltpu.async_copy(h_hbm.at[idx_s.at[r]],
                                            rows_b.at[pl.ds(r * 128, 128)], semf)
                           for r in range(RPB)]
                    cps = [pltpu.async_copy(u_sp.at[idx_s.at[r]], ub.at[r], sems)
                           for r in range(RPB)]
                    cps += [pltpu.async_copy(v_sp.at[idx_d.at[r]], vb.at[r], sems)
                            for r in range(RPB)]
                    for cp in cps:
                        cp.wait()
                    for r in range(RPB):
                        for q in range(8):
                            e = (ub[r, pl.ds(q * 16, 16)]
                                 + vb[r, pl.ds(q * 16, 16)])
                            e = jnp.where(e >= 0.0, e, 0.2 * e)
                            wb[r, pl.ds(q * 16, 16)] = jnp.exp(e - mvec)
                    cpw = []
                    if hi == 0:
                        cpw = [pltpu.async_copy(wb.at[r], s_sp.at[idx_d.at[r]],
                                                sems, add=True)
                               for r in range(RPB)]
                    for cp in cpf:
                        cp.wait()

                    def sc1(j, _):
                        w16 = wb[j // 8, pl.ds((j % 8) * 16, 16)]
                        for k in range(16):
                            rr = j * 16 + k
                            w_s = w16[k]
                            rows_b[rr, pl.ds(0, 16)] = rows_b[rr, pl.ds(0, 16)] * w_s
                            rows_b[rr, pl.ds(16, 16)] = rows_b[rr, pl.ds(16, 16)] * w_s
                        return 0
                    lax.fori_loop(0, KS // 16, sc1, 0)

                    cpo = [pltpu.async_copy(rows_b.at[pl.ds(r * 128, 128)],
                                            out_sp.at[idx_d.at[r]], semf,
                                            add=True)
                           for r in range(RPB)]
                    for cp in cpw:
                        cp.wait()
                    for cp in cpo:
                        cp.wait()
                    return 0

                lax.fori_loop(0, nblk, blk, 0)
                plsc.subcore_barrier()
                for i in range(7):
                    ck = pl.ds(s * STRIPE + i * 448, 448)
                    pltpu.sync_copy(out_sp.at[ck], rows_b.at[pl.ds(0, 448)])
                    pltpu.sync_copy(rows_b.at[pl.ds(0, 448)],
                                    out_hbm.at[hi, c, ck])
                plsc.subcore_barrier()

            for h in range(2):
                sth = pl.ds(s * STRIPE + h * HSTRIPE, HSTRIPE)
                pltpu.sync_copy(s_sp.at[sth], zb1)
                pltpu.sync_copy(zb1, s_hbm.at[c, sth])

        pl.run_scoped(
            phases,
            pltpu.VMEM((RPB, 128), jnp.int32),
            pltpu.VMEM((RPB, 128), jnp.int32),
            pltpu.VMEM((RPB, 128), jnp.float32),
            pltpu.VMEM((RPB, 128), jnp.float32),
            pltpu.VMEM((RPB, 128), jnp.float32),
            pltpu.VMEM((KS, HID), jnp.float32),
            pltpu.VMEM((HSTRIPE,), jnp.float32),
            pltpu.VMEM((16, HID), jnp.float32),
            pltpu.VMEM((16,), jnp.float32),
        )

    return pl.kernel(
        body,
        out_type=(
            jax.ShapeDtypeStruct((n_h, NSC, N_PAD, HID), jnp.float32),
            jax.ShapeDtypeStruct((NSC, N_PAD), jnp.float32),
        ),
        mesh=mesh,
        compiler_params=pltpu.CompilerParams(
            needs_layout_passes=False, use_tc_tiling_on_sc=False),
        scratch_types=[
            pltpu.VMEM_SHARED((N_PAD,), jnp.float32),
            pltpu.VMEM_SHARED((N_PAD,), jnp.float32),
            pltpu.VMEM_SHARED((N_PAD,), jnp.float32),
            pltpu.VMEM_SHARED((N_PAD, HID), jnp.float32),
            pltpu.SemaphoreType.DMA,
            pltpu.SemaphoreType.DMA,
        ],
    )


_sc_agg_pos2 = _make_sc_agg(E_T_POS, 2)
_sc_agg_neg2 = _make_sc_agg(E_T_NEG, 2)
_sc_agg_pos1 = _make_sc_agg(E_T_POS, 1)
_sc_agg_neg1 = _make_sc_agg(E_T_NEG, 1)


def _tc_scores_body(x_ref, a_ref, out_ref, mx_ref):
    i = pl.program_id(0)
    blk = lax.dot_general(
        a_ref[...], x_ref[...], (((1,), (1,)), ((), ())),
        preferred_element_type=jnp.float32)
    out_ref[...] = blk
    bm = jnp.broadcast_to(jnp.max(blk, axis=1)[:, None], (8, 128))

    @pl.when(i == 0)
    def _():
        mx_ref[...] = bm

    @pl.when(i > 0)
    def _():
        mx_ref[...] = jnp.maximum(mx_ref[...], bm)


def _tc_scores(x, a_mat):
    return pl.pallas_call(
        _tc_scores_body,
        grid=(N_PAD // B_TC,),
        in_specs=[pl.BlockSpec((B_TC, DIN), lambda i: (i, 0)),
                  pl.BlockSpec((8, DIN), lambda i: (0, 0))],
        out_specs=[pl.BlockSpec((8, B_TC), lambda i: (0, i)),
                   pl.BlockSpec((8, 128), lambda i: (0, 0))],
        out_shape=[jax.ShapeDtypeStruct((8, N_PAD), jnp.float32),
                   jax.ShapeDtypeStruct((8, 128), jnp.float32)],
    )(x, a_mat)


def _combine(p_ref, s_ref, n_h):
    den = s_ref[0] + s_ref[1]
    inv = 1.0 / (den + 1e-16)
    parts = [(p_ref[h, 0] + p_ref[h, 1]) * inv[:, None] for h in range(n_h)]
    return jnp.concatenate(parts, axis=1) if n_h > 1 else parts[0]


def _tc_layer1_body(x_ref, pp_ref, sp_ref, pn_ref, sn_ref,
                    w1p_ref, b1p_ref, w1n_ref, b1n_ref, wuv_ref,
                    zp_ref, zn_ref, uv2_ref, mx_ref):
    i = pl.program_id(0)
    x = x_ref[...]
    agg_p = _combine(pp_ref, sp_ref, 2)
    agg_n = _combine(pn_ref, sn_ref, 2)
    hp = jnp.concatenate([agg_p, x], axis=1) @ w1p_ref[...] + b1p_ref[...]
    hn = jnp.concatenate([agg_n, x], axis=1) @ w1n_ref[...] + b1n_ref[...]
    zp = jnp.tanh(hp)
    zn = jnp.tanh(hn)
    zp_ref[...] = zp
    zn_ref[...] = zn
    uv2 = lax.dot_general(
        wuv_ref[...], jnp.concatenate([zp, zn], axis=1),
        (((1,), (1,)), ((), ())), preferred_element_type=jnp.float32)
    uv2_ref[...] = uv2
    bm = jnp.broadcast_to(jnp.max(uv2, axis=1)[:, None], (8, 128))

    @pl.when(i == 0)
    def _():
        mx_ref[...] = bm

    @pl.when(i > 0)
    def _():
        mx_ref[...] = jnp.maximum(mx_ref[...], bm)


def _tc_layer1(x, pp, sp, pn, sn, w1p, b1p, w1n, b1n, wuv):
    part2 = pl.BlockSpec((2, NSC, B_TC, HID), lambda i: (0, 0, i, 0))
    sspec = pl.BlockSpec((NSC, B_TC), lambda i: (0, i))
    return pl.pallas_call(
        _tc_layer1_body,
        grid=(N_PAD // B_TC,),
        in_specs=[pl.BlockSpec((B_TC, DIN), lambda i: (i, 0)),
                  part2, sspec, part2, sspec,
                  pl.BlockSpec((2 * DIN, HID), lambda i: (0, 0)),
                  pl.BlockSpec((HID,), lambda i: (0,)),
                  pl.BlockSpec((2 * DIN, HID), lambda i: (0, 0)),
                  pl.BlockSpec((HID,), lambda i: (0,)),
                  pl.BlockSpec((8, 2 * HID), lambda i: (0, 0))],
        out_specs=[pl.BlockSpec((B_TC, HID), lambda i: (i, 0)),
                   pl.BlockSpec((B_TC, HID), lambda i: (i, 0)),
                   pl.BlockSpec((8, B_TC), lambda i: (0, i)),
                   pl.BlockSpec((8, 128), lambda i: (0, 0))],
        out_shape=[jax.ShapeDtypeStruct((N_PAD, HID), jnp.float32),
                   jax.ShapeDtypeStruct((N_PAD, HID), jnp.float32),
                   jax.ShapeDtypeStruct((8, N_PAD), jnp.float32),
                   jax.ShapeDtypeStruct((8, 128), jnp.float32)],
    )(x, pp, sp, pn, sn, w1p, b1p, w1n, b1n, wuv)


def _tc_layer2_body(zp_ref, zn_ref,
                    ppp_ref, spp_ref, pnn_ref, snn_ref,
                    pnp_ref, snp_ref, ppn_ref, spn_ref,
                    w2p_ref, b2p_ref, w2n_ref, b2n_ref, out_ref):
    agg_pp = _combine(ppp_ref, spp_ref, 1)
    agg_nn = _combine(pnn_ref, snn_ref, 1)
    agg_np = _combine(pnp_ref, snp_ref, 1)
    agg_pn = _combine(ppn_ref, spn_ref, 1)
    op = (jnp.concatenate([agg_pp, agg_nn, zp_ref[...]], axis=1)
          @ w2p_ref[...] + b2p_ref[...])
    on = (jnp.concatenate([agg_np, agg_pn, zn_ref[...]], axis=1)
          @ w2n_ref[...] + b2n_ref[...])
    out_ref[...] = jnp.tanh(jnp.concatenate([op, on], axis=1))


def _tc_layer2(zp, zn, aggs, w2p, b2p, w2n, b2n):
    part1 = pl.BlockSpec((1, NSC, B_TC, HID), lambda i: (0, 0, i, 0))
    sspec = pl.BlockSpec((NSC, B_TC), lambda i: (0, i))
    zspec = pl.BlockSpec((B_TC, HID), lambda i: (i, 0))
    agg_ops = []
    agg_specs = []
    for (p, sv) in aggs:
        agg_ops += [p, sv]
        agg_specs += [part1, sspec]
    return pl.pallas_call(
        _tc_layer2_body,
        grid=(N_PAD // B_TC,),
        in_specs=[zspec, zspec] + agg_specs +
                 [pl.BlockSpec((3 * HID, HID), lambda i: (0, 0)),
                  pl.BlockSpec((HID,), lambda i: (0,)),
                  pl.BlockSpec((3 * HID, HID), lambda i: (0, 0)),
                  pl.BlockSpec((HID,), lambda i: (0,))],
        out_specs=pl.BlockSpec((B_TC, DIN), lambda i: (i, 0)),
        out_shape=jax.ShapeDtypeStruct((N_PAD, DIN), jnp.float32),
    )(zp, zn, *agg_ops, w2p, b2p, w2n, b2n)


def kernel(x, pos_edge_index, neg_edge_index,
           a1_pos, a1_neg, W1_pos, b1_pos, W1_neg, b1_neg,
           a2_pos, a2_neg, W2_pos, b2_pos, W2_neg, b2_neg):
    zeros32 = jnp.zeros((HID,), jnp.float32)

    psrc, pdst = _prep_edges(pos_edge_index, E_T_POS)
    nsrc, ndst = _prep_edges(neg_edge_index, E_T_NEG)
    x0 = x[:, :HID]
    x1 = x[:, HID:]
    a1_mat = jnp.stack([a1_pos[:DIN], a1_pos[DIN:], a1_neg[:DIN], a1_neg[DIN:],
                        jnp.zeros((DIN,), jnp.float32), jnp.zeros((DIN,), jnp.float32),
                        jnp.zeros((DIN,), jnp.float32), jnp.zeros((DIN,), jnp.float32)])
    wuv = jnp.stack([
        jnp.concatenate([a2_pos[:HID], zeros32]),
        jnp.concatenate([a2_pos[HID:], zeros32]),
        jnp.concatenate([zeros32, a2_neg[:HID]]),
        jnp.concatenate([zeros32, a2_neg[HID:]]),
        jnp.concatenate([zeros32, a2_pos[:HID]]),
        jnp.concatenate([zeros32, a2_pos[HID:]]),
        jnp.concatenate([a2_neg[:HID], zeros32]),
        jnp.concatenate([a2_neg[HID:], zeros32]),
    ])

    xp = jnp.zeros((N_PAD, DIN), jnp.float32).at[:N].set(x)
    uv1, mx1 = _tc_scores(xp, a1_mat)
    m1p = mx1[0, :16] + mx1[1, :16]
    m1n = mx1[2, :16] + mx1[3, :16]
    pp, sp = _sc_agg_pos2(m1p, uv1[0], uv1[1], x0, x1, psrc, pdst)
    pn, sn = _sc_agg_neg2(m1n, uv1[2], uv1[3], x0, x1, nsrc, ndst)
    zp, zn, uv2, mx2 = _tc_layer1(xp, pp, sp, pn, sn,
                                  W1_pos, b1_pos, W1_neg, b1_neg, wuv)

    m_pp = mx2[0, :16] + mx2[1, :16]
    m_nn = mx2[2, :16] + mx2[3, :16]
    m_np = mx2[4, :16] + mx2[5, :16]
    m_pn = mx2[6, :16] + mx2[7, :16]
    agg_pp = _sc_agg_pos1(m_pp, uv2[0], uv2[1], zp, psrc, pdst)
    agg_nn = _sc_agg_neg1(m_nn, uv2[2], uv2[3], zn, nsrc, ndst)
    agg_np = _sc_agg_pos1(m_np, uv2[4], uv2[5], zn, psrc, pdst)
    agg_pn = _sc_agg_neg1(m_pn, uv2[6], uv2[7], zp, nsrc, ndst)
    out = _tc_layer2(zp, zn, [agg_pp, agg_nn, agg_np, agg_pn],
                     W2_pos, b2_pos, W2_neg, b2_neg)
    return out[:N]


# speedup vs baseline: 50.0793x; 1.0458x over previous
import functools

import jax
import jax.numpy as jnp
from jax import lax
from jax.experimental import pallas as pl
from jax.experimental.pallas import tpu as pltpu
from jax.experimental.pallas import tpu_sc as plsc

N = 50000
DIN = 64
HID = 32
NSC = 2
NT = 16
N_PAD = 50176
STRIPE = N_PAD // NT
HSTRIPE = STRIPE // 2
KS = 512
E_T_POS = 18944
E_T_NEG = 6656
B_TC = 1792
RPB = KS // 128


def _prep_edges(ei, e_tile):
    e_tot = NSC * NT * e_tile
    e = ei.shape[1]
    pad = e_tot - e
    pad_src = jnp.arange(pad, dtype=jnp.int32) % N
    pad_dst = N + jnp.arange(pad, dtype=jnp.int32) % (N_PAD - N)
    src = jnp.concatenate([ei[0], pad_src])
    dst = jnp.concatenate([ei[1], pad_dst])
    return src.reshape(-1, 128), dst.reshape(-1, 128)


def _make_sc_agg(e_tile, n_h):
    nblk = e_tile // KS
    rows_per_tile = e_tile // 128
    mesh = plsc.VectorSubcoreMesh(core_axis_name="c", subcore_axis_name="s")

    def body(m_hbm, u_hbm, v_hbm, *rest):
        zeros16 = jnp.zeros((16,), jnp.float32)
        h_hbms = rest[:n_h]
        (src_hbm, dst_hbm, out_hbm, s_hbm,
         u_sp, v_sp, s_sp, out_sp, semf, sems) = rest[n_h:]
        c = lax.axis_index("c")
        s = lax.axis_index("s")
        tile_rows = (c * NT + s) * rows_per_tile

        def phases(idx_s, idx_d, ub, vb, wb, rows_b, zb1, zb2, mv):
            st = pl.ds(s * STRIPE, STRIPE)
            for h in range(2):
                sth = pl.ds(s * STRIPE + h * HSTRIPE, HSTRIPE)
                pltpu.sync_copy(u_hbm.at[sth], zb1)
                pltpu.sync_copy(zb1, u_sp.at[sth])
                pltpu.sync_copy(v_hbm.at[sth], zb1)
                pltpu.sync_copy(zb1, v_sp.at[sth])

            pltpu.sync_copy(m_hbm, mv)
            mvec = mv[pl.ds(0, 16)]

            def z1(i, _):
                zb1[pl.ds(i * 16, 16)] = zeros16
                return 0
            lax.fori_loop(0, HSTRIPE // 16, z1, 0, unroll=8)

            def z2(i, _):
                zb2[i, pl.ds(0, 16)] = zeros16
                zb2[i, pl.ds(16, 16)] = zeros16
                return 0
            lax.fori_loop(0, 16, z2, 0, unroll=8)

            for h in range(2):
                pltpu.sync_copy(zb1, s_sp.at[pl.ds(s * STRIPE + h * HSTRIPE,
                                                   HSTRIPE)])
            plsc.subcore_barrier()

            for hi in range(n_h):
                h_hbm = h_hbms[hi]

                def zr(i, _):
                    rows_b[i, pl.ds(0, 16)] = zeros16
                    rows_b[i, pl.ds(16, 16)] = zeros16
                    return 0
                lax.fori_loop(0, KS, zr, 0, unroll=8)

                def z3(i, _):
                    pltpu.sync_copy(rows_b,
                                    out_sp.at[pl.ds(s * STRIPE + i * KS, KS)])
                    return 0
                lax.fori_loop(0, STRIPE // KS, z3, 0)
                rem = STRIPE - KS * (STRIPE // KS)
                pltpu.sync_copy(
                    rows_b.at[pl.ds(0, rem)],
                    out_sp.at[pl.ds(s * STRIPE + KS * (STRIPE // KS), rem)])
                plsc.subcore_barrier()

                def blk(b, _):
                    row0 = tile_rows + b * RPB
                    pltpu.sync_copy(src_hbm.at[pl.ds(row0, RPB)], idx_s)
                    pltpu.sync_copy(dst_hbm.at[pl.ds(row0, RPB)], idx_d)
                    cpf = [pltpu.async_copy(h_hbm.at[idx_s.at[r]],
                                            rows_b.at[pl.ds(r * 128, 128)], semf)
                           for r in range(RPB)]
                    cps = [pltpu.async_copy(u_sp.at[idx_s.at[r]], ub.at[r], sems)
                           for r in range(RPB)]
                    cps += [pltpu.async_copy(v_sp.at[idx_d.at[r]], vb.at[r], sems)
                            for r in range(RPB)]
                    for cp in cps:
                        cp.wait()
                    for r in range(RPB):
                        for q in range(8):
                            e = (ub[r, pl.ds(q * 16, 16)]
                                 + vb[r, pl.ds(q * 16, 16)])
                            e = jnp.where(e >= 0.0, e, 0.2 * e)
                            wb[r, pl.ds(q * 16, 16)] = jnp.exp(e - mvec)
                    cpw = []
                    if hi == 0:
                        cpw = [pltpu.async_copy(wb.at[r], s_sp.at[idx_d.at[r]],
                                                sems, add=True)
                               for r in range(RPB)]
                    for cp in cpf:
                        cp.wait()

                    def sc1(j, _):
                        w16 = wb[j // 8, pl.ds((j % 8) * 16, 16)]
                        for k in range(16):
                            rr = j * 16 + k
                            w_s = w16[k]
                            rows_b[rr, pl.ds(0, 16)] = rows_b[rr, pl.ds(0, 16)] * w_s
                            rows_b[rr, pl.ds(16, 16)] = rows_b[rr, pl.ds(16, 16)] * w_s
                        return 0
                    lax.fori_loop(0, KS // 16, sc1, 0)

                    cpo = [pltpu.async_copy(rows_b.at[pl.ds(r * 128, 128)],
                                            out_sp.at[idx_d.at[r]], semf,
                                            add=True)
                           for r in range(RPB)]
                    for cp in cpw:
                        cp.wait()
                    for cp in cpo:
                        cp.wait()
                    return 0

                lax.fori_loop(0, nblk, blk, 0)
                plsc.subcore_barrier()
                for i in range(7):
                    ck = pl.ds(s * STRIPE + i * 448, 448)
                    pltpu.sync_copy(out_sp.at[ck], rows_b.at[pl.ds(0, 448)])
                    pltpu.sync_copy(rows_b.at[pl.ds(0, 448)],
                                    out_hbm.at[hi, c, ck])
                plsc.subcore_barrier()

            for h in range(2):
                sth = pl.ds(s * STRIPE + h * HSTRIPE, HSTRIPE)
                pltpu.sync_copy(s_sp.at[sth], zb1)
                pltpu.sync_copy(zb1, s_hbm.at[c, sth])

        pl.run_scoped(
            phases,
            pltpu.VMEM((RPB, 128), jnp.int32),
            pltpu.VMEM((RPB, 128), jnp.int32),
            pltpu.VMEM((RPB, 128), jnp.float32),
            pltpu.VMEM((RPB, 128), jnp.float32),
            pltpu.VMEM((RPB, 128), jnp.float32),
            pltpu.VMEM((KS, HID), jnp.float32),
            pltpu.VMEM((HSTRIPE,), jnp.float32),
            pltpu.VMEM((16, HID), jnp.float32),
            pltpu.VMEM((16,), jnp.float32),
        )

    return pl.kernel(
        body,
        out_type=(
            jax.ShapeDtypeStruct((n_h, NSC, N_PAD, HID), jnp.float32),
            jax.ShapeDtypeStruct((NSC, N_PAD), jnp.float32),
        ),
        mesh=mesh,
        compiler_params=pltpu.CompilerParams(
            needs_layout_passes=False, use_tc_tiling_on_sc=False),
        scratch_types=[
            pltpu.VMEM_SHARED((N_PAD,), jnp.float32),
            pltpu.VMEM_SHARED((N_PAD,), jnp.float32),
            pltpu.VMEM_SHARED((N_PAD,), jnp.float32),
            pltpu.VMEM_SHARED((N_PAD, HID), jnp.float32),
            pltpu.SemaphoreType.DMA,
            pltpu.SemaphoreType.DMA,
        ],
    )


_sc_agg_pos2 = _make_sc_agg(E_T_POS, 2)
_sc_agg_neg2 = _make_sc_agg(E_T_NEG, 2)
_sc_agg_pos1 = _make_sc_agg(E_T_POS, 1)
_sc_agg_neg1 = _make_sc_agg(E_T_NEG, 1)


def _tc_scores_body(x_ref, a_ref, out_ref, mx_ref):
    i = pl.program_id(0)
    blk = lax.dot_general(
        a_ref[...], x_ref[...], (((1,), (1,)), ((), ())),
        preferred_element_type=jnp.float32)
    out_ref[...] = blk
    bm = jnp.broadcast_to(jnp.max(blk, axis=1)[:, None], (8, 128))

    @pl.when(i == 0)
    def _():
        mx_ref[...] = bm

    @pl.when(i > 0)
    def _():
        mx_ref[...] = jnp.maximum(mx_ref[...], bm)


def _tc_scores(x, a_mat):
    return pl.pallas_call(
        _tc_scores_body,
        grid=(N_PAD // B_TC,),
        in_specs=[pl.BlockSpec((B_TC, DIN), lambda i: (i, 0)),
                  pl.BlockSpec((8, DIN), lambda i: (0, 0))],
        out_specs=[pl.BlockSpec((8, B_TC), lambda i: (0, i)),
                   pl.BlockSpec((8, 128), lambda i: (0, 0))],
        out_shape=[jax.ShapeDtypeStruct((8, N_PAD), jnp.float32),
                   jax.ShapeDtypeStruct((8, 128), jnp.float32)],
    )(x, a_mat)


def _combine(p_ref, s_ref, n_h):
    den = s_ref[0] + s_ref[1]
    inv = 1.0 / (den + 1e-16)
    parts = [(p_ref[h, 0] + p_ref[h, 1]) * inv[:, None] for h in range(n_h)]
    return jnp.concatenate(parts, axis=1) if n_h > 1 else parts[0]


def _tc_layer1_body(x_ref, pp_ref, sp_ref, pn_ref, sn_ref,
                    w1p_ref, b1p_ref, w1n_ref, b1n_ref, wuv_ref,
                    zp_ref, zn_ref, uv2_ref, mx_ref):
    i = pl.program_id(0)
    x = x_ref[...]
    agg_p = _combine(pp_ref, sp_ref, 2)
    agg_n = _combine(pn_ref, sn_ref, 2)
    hp = jnp.concatenate([agg_p, x], axis=1) @ w1p_ref[...] + b1p_ref[...]
    hn = jnp.concatenate([agg_n, x], axis=1) @ w1n_ref[...] + b1n_ref[...]
    zp = jnp.tanh(hp)
    zn = jnp.tanh(hn)
    zp_ref[...] = zp
    zn_ref[...] = zn
    uv2 = lax.dot_general(
        wuv_ref[...], jnp.concatenate([zp, zn], axis=1),
        (((1,), (1,)), ((), ())), preferred_element_type=jnp.float32)
    uv2_ref[...] = uv2
    bm = jnp.broadcast_to(jnp.max(uv2, axis=1)[:, None], (8, 128))

    @pl.when(i == 0)
    def _():
        mx_ref[...] = bm

    @pl.when(i > 0)
    def _():
        mx_ref[...] = jnp.maximum(mx_ref[...], bm)


def _tc_layer1(x, pp, sp, pn, sn, w1p, b1p, w1n, b1n, wuv):
    part2 = pl.BlockSpec((2, NSC, B_TC, HID), lambda i: (0, 0, i, 0))
    sspec = pl.BlockSpec((NSC, B_TC), lambda i: (0, i))
    return pl.pallas_call(
        _tc_layer1_body,
        grid=(N_PAD // B_TC,),
        in_specs=[pl.BlockSpec((B_TC, DIN), lambda i: (i, 0)),
                  part2, sspec, part2, sspec,
                  pl.BlockSpec((2 * DIN, HID), lambda i: (0, 0)),
                  pl.BlockSpec((HID,), lambda i: (0,)),
                  pl.BlockSpec((2 * DIN, HID), lambda i: (0, 0)),
                  pl.BlockSpec((HID,), lambda i: (0,)),
                  pl.BlockSpec((8, 2 * HID), lambda i: (0, 0))],
        out_specs=[pl.BlockSpec((B_TC, HID), lambda i: (i, 0)),
                   pl.BlockSpec((B_TC, HID), lambda i: (i, 0)),
                   pl.BlockSpec((8, B_TC), lambda i: (0, i)),
                   pl.BlockSpec((8, 128), lambda i: (0, 0))],
        out_shape=[jax.ShapeDtypeStruct((N_PAD, HID), jnp.float32),
                   jax.ShapeDtypeStruct((N_PAD, HID), jnp.float32),
                   jax.ShapeDtypeStruct((8, N_PAD), jnp.float32),
                   jax.ShapeDtypeStruct((8, 128), jnp.float32)],
    )(x, pp, sp, pn, sn, w1p, b1p, w1n, b1n, wuv)


def _tc_layer2_body(zp_ref, zn_ref,
                    ppp_ref, spp_ref, pnn_ref, snn_ref,
                    pnp_ref, snp_ref, ppn_ref, spn_ref,
                    w2p_ref, b2p_ref, w2n_ref, b2n_ref, out_ref):
    agg_pp = _combine(ppp_ref, spp_ref, 1)
    agg_nn = _combine(pnn_ref, snn_ref, 1)
    agg_np = _combine(pnp_ref, snp_ref, 1)
    agg_pn = _combine(ppn_ref, spn_ref, 1)
    op = (jnp.concatenate([agg_pp, agg_nn, zp_ref[...]], axis=1)
          @ w2p_ref[...] + b2p_ref[...])
    on = (jnp.concatenate([agg_np, agg_pn, zn_ref[...]], axis=1)
          @ w2n_ref[...] + b2n_ref[...])
    out_ref[...] = jnp.tanh(jnp.concatenate([op, on], axis=1))


def _tc_layer2(zp, zn, aggs, w2p, b2p, w2n, b2n):
    part1 = pl.BlockSpec((1, NSC, B_TC, HID), lambda i: (0, 0, i, 0))
    sspec = pl.BlockSpec((NSC, B_TC), lambda i: (0, i))
    zspec = pl.BlockSpec((B_TC, HID), lambda i: (i, 0))
    agg_ops = []
    agg_specs = []
    for (p, sv) in aggs:
        agg_ops += [p, sv]
        agg_specs += [part1, sspec]
    return pl.pallas_call(
        _tc_layer2_body,
        grid=(N_PAD // B_TC,),
        in_specs=[zspec, zspec] + agg_specs +
                 [pl.BlockSpec((3 * HID, HID), lambda i: (0, 0)),
                  pl.BlockSpec((HID,), lambda i: (0,)),
                  pl.BlockSpec((3 * HID, HID), lambda i: (0, 0)),
                  pl.BlockSpec((HID,), lambda i: (0,))],
        out_specs=pl.BlockSpec((B_TC, DIN), lambda i: (i, 0)),
        out_shape=jax.ShapeDtypeStruct((N_PAD, DIN), jnp.float32),
    )(zp, zn, *agg_ops, w2p, b2p, w2n, b2n)


def kernel(x, pos_edge_index, neg_edge_index,
           a1_pos, a1_neg, W1_pos, b1_pos, W1_neg, b1_neg,
           a2_pos, a2_neg, W2_pos, b2_pos, W2_neg, b2_neg):
    zeros32 = jnp.zeros((HID,), jnp.float32)

    psrc, pdst = _prep_edges(pos_edge_index, E_T_POS)
    nsrc, ndst = _prep_edges(neg_edge_index, E_T_NEG)
    x0 = x[:, :HID]
    x1 = x[:, HID:]
    a1_mat = jnp.stack([a1_pos[:DIN], a1_pos[DIN:], a1_neg[:DIN], a1_neg[DIN:],
                        jnp.zeros((DIN,), jnp.float32), jnp.zeros((DIN,), jnp.float32),
                        jnp.zeros((DIN,), jnp.float32), jnp.zeros((DIN,), jnp.float32)])
    wuv = jnp.stack([
        jnp.concatenate([a2_pos[:HID], zeros32]),
        jnp.concatenate([a2_pos[HID:], zeros32]),
        jnp.concatenate([zeros32, a2_neg[:HID]]),
        jnp.concatenate([zeros32, a2_neg[HID:]]),
        jnp.concatenate([zeros32, a2_pos[:HID]]),
        jnp.concatenate([zeros32, a2_pos[HID:]]),
        jnp.concatenate([a2_neg[:HID], zeros32]),
        jnp.concatenate([a2_neg[HID:], zeros32]),
    ])

    xp = jnp.zeros((N_PAD, DIN), jnp.float32).at[:N].set(x)
    uv1, mx1 = _tc_scores(xp, a1_mat)
    m1p = mx1[0, :16] + mx1[1, :16]
    m1n = mx1[2, :16] + mx1[3, :16]
    pp, sp = _sc_agg_pos2(m1p, uv1[0], uv1[1], x0, x1, psrc, pdst)
    pn, sn = _sc_agg_neg2(m1n, uv1[2], uv1[3], x0, x1, nsrc, ndst)
    zp, zn, uv2, mx2 = _tc_layer1(xp, pp, sp, pn, sn,
                                  W1_pos, b1_pos, W1_neg, b1_neg, wuv)

    m_pp = mx2[0, :16] + mx2[1, :16]
    m_nn = mx2[2, :16] + mx2[3, :16]
    m_np = mx2[4, :16] + mx2[5, :16]
    m_pn = mx2[6, :16] + mx2[7, :16]
    agg_pp = _sc_agg_pos1(m_pp, uv2[0], uv2[1], zp, psrc, pdst)
    agg_nn = _sc_agg_neg1(m_nn, uv2[2], uv2[3], zn, nsrc, ndst)
    agg_np = _sc_agg_pos1(m_np, uv2[4], uv2[5], zn, psrc, pdst)
    agg_pn = _sc_agg_neg1(m_pn, uv2[6], uv2[7], zp, nsrc, ndst)
    out = _tc_layer2(zp, zn, [agg_pp, agg_nn, agg_np, agg_pn],
                     W2_pos, b2_pos, W2_neg, b2_neg)
    return out[:N]
